# baseline (speedup 1.0000x reference)
import jax
import jax.numpy as jnp
from jax.experimental import pallas as pl
from jax.experimental.pallas import tpu as pltpu

_E = 64
_BIG = 1e30


def _chamfer_kernel(f_ref, ip_ref, op_ref, nz_ref, z_ref):
    i = pl.program_id(0)
    f32 = jnp.float32

    ya = f_ref[:, 0:6, :]
    xa = f_ref[:, 6:12, :]
    xb = f_ref[:, 12:18, :]
    yb = f_ref[:, 18:24, :]
    x2 = xa[:, 4:5, :]
    y2 = yb[:, 4:5, :]
    in_row = (ip_ref[...] != 0).astype(f32)
    out_row = (op_ref[...] != 0).astype(f32)

    tn = (((1,), (1,)), ((0,), (0,)))
    m1 = jax.lax.dot_general(ya, xa, tn, preferred_element_type=f32)
    m2 = jax.lax.dot_general(xb, yb, tn, preferred_element_type=f32)

    min_xy = jnp.sqrt(jnp.maximum(jnp.min(m1, axis=1, keepdims=True), 0.0))
    min_yx = jnp.sqrt(jnp.maximum(jnp.min(m2, axis=1, keepdims=True), 0.0))

    cnt_in = jnp.sum(in_row, axis=2, keepdims=True)
    cnt_out = jnp.sum(out_row, axis=2, keepdims=True)
    n_in = jnp.maximum(1.0, cnt_in)
    n_out = jnp.maximum(1.0, cnt_out)

    sum_xy = jnp.sum(in_row * min_xy, axis=2, keepdims=True)
    sum_yx = jnp.sum(out_row * min_yx, axis=2, keepdims=True)
    e_both = 0.5 * (sum_xy / n_out + sum_yx / n_in)

    x_norm = jnp.sqrt(x2)
    y_norm = jnp.sqrt(y2)
    x_norm_sum = jnp.sum(in_row * x_norm, axis=2, keepdims=True)
    e_nz = jnp.where(cnt_out == 0.0, x_norm_sum / n_in,
                     jnp.where(cnt_in == 0.0, x_norm_sum / n_out, e_both))

    n_oz = jnp.maximum(1.0, 256.0 - cnt_out)
    e_z = jnp.sum((1.0 - out_row) * y_norm, axis=2, keepdims=True) / n_oz

    @pl.when(i == 0)
    def _init():
        nz_ref[0, 0] = 0.0
        z_ref[0, 0] = 0.0

    nz_ref[0, 0] += jnp.sum(e_nz)
    z_ref[0, 0] += jnp.sum(e_z)


def kernel(target, reco, in_pid, out_pid):
    n_batches = target.shape[0]
    n_steps = n_batches // _E
    rt2 = 1.4142135623730951

    xt = target.transpose(0, 2, 1)
    yt = reco.transpose(0, 2, 1)
    ip3 = in_pid.reshape(n_batches, 1, 256)
    op3 = out_pid.reshape(n_batches, 1, 256)
    x2 = jnp.sum(xt * xt, axis=1, keepdims=True)
    y2 = jnp.sum(yt * yt, axis=1, keepdims=True)
    pen_in = jnp.where(ip3 != 0, 0.0, _BIG)
    pen_out = jnp.where(op3 != 0, 0.0, _BIG)
    ones = jnp.ones_like(x2)
    feats = jnp.concatenate([
        -rt2 * yt, ones, y2 + pen_out,
        rt2 * xt, x2, ones,
        -rt2 * xt, ones, x2 + pen_in,
        rt2 * yt, y2, ones,
    ], axis=1)

    nz, z = pl.pallas_call(
        _chamfer_kernel,
        grid=(n_steps,),
        in_specs=[
            pl.BlockSpec((_E, 24, 256), lambda i: (i, 0, 0)),
            pl.BlockSpec((_E, 1, 256), lambda i: (i, 0, 0)),
            pl.BlockSpec((_E, 1, 256), lambda i: (i, 0, 0)),
        ],
        out_specs=[
            pl.BlockSpec(memory_space=pltpu.SMEM),
            pl.BlockSpec(memory_space=pltpu.SMEM),
        ],
        out_shape=[
            jax.ShapeDtypeStruct((1, 1), jnp.float32),
            jax.ShapeDtypeStruct((1, 1), jnp.float32),
        ],
        compiler_params=pltpu.CompilerParams(
            dimension_semantics=("arbitrary",)),
    )(feats, ip3, op3)

    inv = 1.0 / n_batches
    return (nz * inv).reshape(()), (z * inv).reshape(())
